# Initial kernel scaffold; baseline (speedup 1.0000x reference)
import jax
import jax.numpy as jnp
from jax.experimental import pallas as pl


def kernel(x, W):
    raise NotImplementedError("write your pallas kernel here")


# speedup vs baseline: 3.0776x; 3.0776x over previous
import functools

import jax
import jax.numpy as jnp
from jax import lax
from jax.experimental import pallas as pl
from jax.experimental.pallas import tpu as pltpu
from jax.experimental.pallas import tpu_sc as plsc

ROWS, COLS = 4096, 50
EMBED_DIM = 128
B = ROWS * COLS

NUM_CORES = 2
NUM_SUBCORES = 16
NW = NUM_CORES * NUM_SUBCORES
B_PER_W = B // NW
CHUNK = 128
NCHUNK = B_PER_W // CHUNK


def _embed_body(x_hbm, w_hbm, out_hbm, idx_v, rows_v, sem):
    wid = lax.axis_index("s") * NUM_CORES + lax.axis_index("c")
    base = wid * B_PER_W
    pltpu.sync_copy(x_hbm.at[pl.ds(base, B_PER_W)], idx_v)

    def step(i, carry):
        off = i * CHUNK
        pltpu.async_copy(
            w_hbm.at[idx_v.at[pl.ds(off, CHUNK)]], rows_v, sem
        ).wait()
        pltpu.sync_copy(rows_v, out_hbm.at[pl.ds(base + off, CHUNK)])
        return carry

    lax.fori_loop(0, NCHUNK, step, 0)


@functools.partial(jax.jit, static_argnames=())
def _embed(x_flat, W):
    mesh = plsc.VectorSubcoreMesh(core_axis_name="c", subcore_axis_name="s")
    run = pl.kernel(
        _embed_body,
        mesh=mesh,
        out_type=jax.ShapeDtypeStruct((B, EMBED_DIM), jnp.float32),
        scratch_types=[
            pltpu.VMEM((B_PER_W,), jnp.int32),
            pltpu.VMEM((CHUNK, EMBED_DIM), jnp.float32),
            pltpu.SemaphoreType.DMA,
        ],
    )
    return run(x_flat, W)


def kernel(x, W):
    x_flat = x.reshape(B).astype(jnp.int32)
    out = _embed(x_flat, W)
    return out.reshape(ROWS, COLS, EMBED_DIM)


# speedup vs baseline: 3.4408x; 1.1180x over previous
import functools

import jax
import jax.numpy as jnp
from jax import lax
from jax.experimental import pallas as pl
from jax.experimental.pallas import tpu as pltpu
from jax.experimental.pallas import tpu_sc as plsc

ROWS, COLS = 4096, 50
EMBED_DIM = 128
B = ROWS * COLS

NUM_CORES = 2
NUM_SUBCORES = 16
NW = NUM_CORES * NUM_SUBCORES
B_PER_W = B // NW
CHUNK = 128
NCHUNK = B_PER_W // CHUNK


def _embed_body(x_hbm, w_hbm, out_hbm, idx_v, rows0, rows1, sem0, sem1):
    wid = lax.axis_index("s") * NUM_CORES + lax.axis_index("c")
    base = wid * B_PER_W
    pltpu.sync_copy(x_hbm.at[pl.ds(base, B_PER_W)], idx_v)

    def gather(j, buf, sem):
        pltpu.async_copy(w_hbm.at[idx_v.at[pl.ds(j * CHUNK, CHUNK)]], buf, sem)

    def wait(buf, sem):
        pltpu.make_async_copy(w_hbm.at[pl.ds(0, CHUNK)], buf, sem).wait()

    def store(j, buf):
        pltpu.sync_copy(buf, out_hbm.at[pl.ds(base + j * CHUNK, CHUNK)])

    gather(0, rows0, sem0)
    gather(1, rows1, sem1)

    def pair(g, carry):
        j = 2 * g
        wait(rows0, sem0)
        store(j, rows0)
        gather(j + 2, rows0, sem0)
        wait(rows1, sem1)
        store(j + 1, rows1)
        gather(j + 3, rows1, sem1)
        return carry

    lax.fori_loop(0, NCHUNK // 2 - 1, pair, 0)
    j_last = NCHUNK - 2
    wait(rows0, sem0)
    store(j_last, rows0)
    wait(rows1, sem1)
    store(j_last + 1, rows1)


@functools.partial(jax.jit, static_argnames=())
def _embed(x_flat, W):
    mesh = plsc.VectorSubcoreMesh(core_axis_name="c", subcore_axis_name="s")
    run = pl.kernel(
        _embed_body,
        mesh=mesh,
        out_type=jax.ShapeDtypeStruct((B, EMBED_DIM), jnp.float32),
        scratch_types=[
            pltpu.VMEM((B_PER_W,), jnp.int32),
            pltpu.VMEM((CHUNK, EMBED_DIM), jnp.float32),
            pltpu.VMEM((CHUNK, EMBED_DIM), jnp.float32),
            pltpu.SemaphoreType.DMA,
            pltpu.SemaphoreType.DMA,
        ],
    )
    return run(x_flat, W)


def kernel(x, W):
    x_flat = x.reshape(B).astype(jnp.int32)
    out = _embed(x_flat, W)
    return out.reshape(ROWS, COLS, EMBED_DIM)


# speedup vs baseline: 3.4556x; 1.0043x over previous
import functools

import jax
import jax.numpy as jnp
from jax import lax
from jax.experimental import pallas as pl
from jax.experimental.pallas import tpu as pltpu
from jax.experimental.pallas import tpu_sc as plsc

ROWS, COLS = 4096, 50
EMBED_DIM = 128
B = ROWS * COLS

NUM_CORES = 2
NUM_SUBCORES = 16
NW = NUM_CORES * NUM_SUBCORES
B_PER_W = B // NW
CHUNK = 400
NCHUNK = B_PER_W // CHUNK


def _embed_body(x_hbm, w_hbm, out_hbm, idx_v, rows0, rows1, sem0, sem1):
    wid = lax.axis_index("s") * NUM_CORES + lax.axis_index("c")
    base = wid * B_PER_W
    pltpu.sync_copy(x_hbm.at[pl.ds(base, B_PER_W)], idx_v)

    def gather(j, buf, sem):
        pltpu.async_copy(w_hbm.at[idx_v.at[pl.ds(j * CHUNK, CHUNK)]], buf, sem)

    def wait(buf, sem):
        pltpu.make_async_copy(w_hbm.at[pl.ds(0, CHUNK)], buf, sem).wait()

    def store(j, buf):
        pltpu.sync_copy(buf, out_hbm.at[pl.ds(base + j * CHUNK, CHUNK)])

    gather(0, rows0, sem0)
    gather(1, rows1, sem1)

    def pair(g, carry):
        j = 2 * g
        wait(rows0, sem0)
        store(j, rows0)
        gather(j + 2, rows0, sem0)
        wait(rows1, sem1)
        store(j + 1, rows1)
        gather(j + 3, rows1, sem1)
        return carry

    lax.fori_loop(0, NCHUNK // 2 - 1, pair, 0)
    j_last = NCHUNK - 2
    wait(rows0, sem0)
    store(j_last, rows0)
    wait(rows1, sem1)
    store(j_last + 1, rows1)


@functools.partial(jax.jit, static_argnames=())
def _embed(x_flat, W):
    mesh = plsc.VectorSubcoreMesh(core_axis_name="c", subcore_axis_name="s")
    run = pl.kernel(
        _embed_body,
        mesh=mesh,
        out_type=jax.ShapeDtypeStruct((B, EMBED_DIM), jnp.float32),
        scratch_types=[
            pltpu.VMEM((B_PER_W,), jnp.int32),
            pltpu.VMEM((CHUNK, EMBED_DIM), jnp.float32),
            pltpu.VMEM((CHUNK, EMBED_DIM), jnp.float32),
            pltpu.SemaphoreType.DMA,
            pltpu.SemaphoreType.DMA,
        ],
    )
    return run(x_flat, W)


def kernel(x, W):
    x_flat = x.reshape(B).astype(jnp.int32)
    out = _embed(x_flat, W)
    return out.reshape(ROWS, COLS, EMBED_DIM)


# speedup vs baseline: 6.1820x; 1.7890x over previous
import functools

import jax
import jax.numpy as jnp
from jax import lax
from jax.experimental import pallas as pl
from jax.experimental.pallas import tpu as pltpu
from jax.experimental.pallas import tpu_sc as plsc

ROWS, COLS = 4096, 50
EMBED_DIM = 128

NUM_CORES = 2
NUM_SUBCORES = 16
NW = NUM_CORES * NUM_SUBCORES
I_PER_W = ROWS // NW
G = 8
NGROUP = I_PER_W // G


def _embed_body(x_hbm, w_hbm, out_hbm, idx_v, rows0, rows1, sem0, sem1):
    wid = lax.axis_index("s") * NUM_CORES + lax.axis_index("c")
    i_base = wid * I_PER_W
    pltpu.sync_copy(x_hbm.at[pl.ds(i_base, I_PER_W)], idx_v)

    def fire(g, buf, sem):
        for ii in range(G):
            pltpu.async_copy(w_hbm.at[idx_v.at[g * G + ii]], buf.at[ii], sem)

    def drain(buf, sem):
        pltpu.make_async_copy(out_hbm.at[pl.ds(0, G)], buf, sem).wait()

    def store(g, buf):
        pltpu.sync_copy(buf, out_hbm.at[pl.ds(i_base + g * G, G)])

    fire(0, rows0, sem0)
    fire(1, rows1, sem1)

    def pair(p, carry):
        g = 2 * p
        drain(rows0, sem0)
        store(g, rows0)
        fire(g + 2, rows0, sem0)
        drain(rows1, sem1)
        store(g + 1, rows1)
        fire(g + 3, rows1, sem1)
        return carry

    lax.fori_loop(0, NGROUP // 2 - 1, pair, 0)
    g_last = NGROUP - 2
    drain(rows0, sem0)
    store(g_last, rows0)
    drain(rows1, sem1)
    store(g_last + 1, rows1)


@jax.jit
def _embed(x, W):
    mesh = plsc.VectorSubcoreMesh(core_axis_name="c", subcore_axis_name="s")
    run = pl.kernel(
        _embed_body,
        mesh=mesh,
        out_type=jax.ShapeDtypeStruct((ROWS, COLS, EMBED_DIM), jnp.float32),
        scratch_types=[
            pltpu.VMEM((I_PER_W, COLS), jnp.int32),
            pltpu.VMEM((G, COLS, EMBED_DIM), jnp.float32),
            pltpu.VMEM((G, COLS, EMBED_DIM), jnp.float32),
            pltpu.SemaphoreType.DMA,
            pltpu.SemaphoreType.DMA,
        ],
    )
    return run(x, W)


def kernel(x, W):
    return _embed(x.astype(jnp.int32), W)


# speedup vs baseline: 10.8095x; 1.7486x over previous
import jax
import jax.numpy as jnp
from jax import lax
from jax.experimental import pallas as pl
from jax.experimental.pallas import tpu as pltpu
from jax.experimental.pallas import tpu_sc as plsc

ROWS, COLS = 4096, 50
EMBED_DIM = 128
B = ROWS * COLS

NUM_CORES = 2
NUM_SUBCORES = 16
NW = NUM_CORES * NUM_SUBCORES
B_PER_W = B // NW
CHUNK = 400
NCHUNK = B_PER_W // CHUNK


def _embed_body(x_hbm, w_hbm, out_hbm, idx_v, rows0, rows1, sem0, sem1):
    wid = lax.axis_index("s") * NUM_CORES + lax.axis_index("c")
    base = wid * B_PER_W
    pltpu.sync_copy(x_hbm.at[pl.ds(base, B_PER_W)], idx_v)

    def gather(j, buf, sem):
        pltpu.async_copy(w_hbm.at[idx_v.at[pl.ds(j * CHUNK, CHUNK)]], buf, sem)

    def wait(buf, sem):
        pltpu.make_async_copy(w_hbm.at[pl.ds(0, CHUNK)], buf, sem).wait()

    def store(j, buf):
        pltpu.sync_copy(buf, out_hbm.at[pl.ds(base + j * CHUNK, CHUNK)])

    gather(0, rows0, sem0)
    gather(1, rows1, sem1)

    def pair(g, carry):
        j = 2 * g
        wait(rows0, sem0)
        store(j, rows0)
        gather(j + 2, rows0, sem0)
        wait(rows1, sem1)
        store(j + 1, rows1)
        gather(j + 3, rows1, sem1)
        return carry

    lax.fori_loop(0, NCHUNK // 2 - 1, pair, 0)
    j_last = NCHUNK - 2
    wait(rows0, sem0)
    store(j_last, rows0)
    wait(rows1, sem1)
    store(j_last + 1, rows1)


@jax.jit
def _embed(x_flat, W):
    mesh = plsc.VectorSubcoreMesh(core_axis_name="c", subcore_axis_name="s")
    run = pl.kernel(
        _embed_body,
        mesh=mesh,
        out_type=jax.ShapeDtypeStruct((B, EMBED_DIM), jnp.float32),
        scratch_types=[
            pltpu.VMEM((B_PER_W,), jnp.int32),
            pltpu.VMEM((CHUNK, EMBED_DIM), jnp.float32),
            pltpu.VMEM((CHUNK, EMBED_DIM), jnp.float32),
            pltpu.SemaphoreType.DMA,
            pltpu.SemaphoreType.DMA,
        ],
    )
    return run(x_flat, W)


def kernel(x, W):
    x_flat = jnp.swapaxes(x, 0, 1).reshape(B).astype(jnp.int32)
    out = _embed(x_flat, W)
    return jnp.swapaxes(out.reshape(COLS, ROWS, EMBED_DIM), 0, 1)


# speedup vs baseline: 10.8788x; 1.0064x over previous
import jax
import jax.numpy as jnp
from jax import lax
from jax.experimental import pallas as pl
from jax.experimental.pallas import tpu as pltpu
from jax.experimental.pallas import tpu_sc as plsc

ROWS, COLS = 4096, 50
EMBED_DIM = 128
B = ROWS * COLS

NUM_CORES = 2
NUM_SUBCORES = 16
NW = NUM_CORES * NUM_SUBCORES
B_PER_W = B // NW
CHUNK = 200
NCHUNK = B_PER_W // CHUNK
NBUF = 4
K = 2
NGROUP = NCHUNK // NBUF


def _embed_body(x_hbm, w_hbm, out_hbm, idx_v, *rest):
    bufs = rest[:NBUF]
    gsem = rest[NBUF : 2 * NBUF]
    ssem = rest[2 * NBUF : 3 * NBUF]

    wid = lax.axis_index("s") * NUM_CORES + lax.axis_index("c")
    base = wid * B_PER_W
    pltpu.sync_copy(x_hbm.at[pl.ds(base, B_PER_W)], idx_v)

    def gather(j, b):
        pltpu.async_copy(
            w_hbm.at[idx_v.at[pl.ds(j * CHUNK, CHUNK)]], bufs[b], gsem[b]
        )

    def wait_gather(b):
        pltpu.make_async_copy(w_hbm.at[pl.ds(0, CHUNK)], bufs[b], gsem[b]).wait()

    def store(j, b):
        pltpu.async_copy(bufs[b], out_hbm.at[pl.ds(base + j * CHUNK, CHUNK)], ssem[b])

    def wait_store(b):
        pltpu.make_async_copy(w_hbm.at[pl.ds(0, CHUNK)], bufs[b], ssem[b]).wait()

    for j in range(K):
        gather(j, j % NBUF)

    for v in range(NBUF):
        b2 = (v + K) % NBUF
        if v + K >= NBUF:
            wait_store(b2)
        gather(v + K, b2)
        wait_gather(v % NBUF)
        store(v, v % NBUF)

    def group(g, carry):
        j0 = g * NBUF
        for v in range(NBUF):
            j = j0 + v
            b = v % NBUF
            b2 = (v + K) % NBUF
            wait_store(b2)
            gather(j + K, b2)
            wait_gather(b)
            store(j, b)
        return carry

    lax.fori_loop(1, NGROUP - 1, group, 0)

    j0 = (NGROUP - 1) * NBUF
    for v in range(NBUF):
        j = j0 + v
        b = v % NBUF
        if j + K < NCHUNK:
            b2 = (v + K) % NBUF
            wait_store(b2)
            gather(j + K, b2)
        wait_gather(b)
        store(j, b)

    for b in range(NBUF):
        wait_store(b)


@jax.jit
def _embed(x_flat, W):
    mesh = plsc.VectorSubcoreMesh(core_axis_name="c", subcore_axis_name="s")
    run = pl.kernel(
        _embed_body,
        mesh=mesh,
        out_type=jax.ShapeDtypeStruct((B, EMBED_DIM), jnp.float32),
        scratch_types=[
            pltpu.VMEM((B_PER_W,), jnp.int32),
            *[pltpu.VMEM((CHUNK, EMBED_DIM), jnp.float32) for _ in range(NBUF)],
            *[pltpu.SemaphoreType.DMA for _ in range(2 * NBUF)],
        ],
    )
    return run(x_flat, W)


def kernel(x, W):
    x_flat = jnp.swapaxes(x, 0, 1).reshape(B).astype(jnp.int32)
    out = _embed(x_flat, W)
    return jnp.swapaxes(out.reshape(COLS, ROWS, EMBED_DIM), 0, 1)
